# Initial kernel scaffold; baseline (speedup 1.0000x reference)
import jax
import jax.numpy as jnp
from jax.experimental import pallas as pl


def kernel(features, W0, b0, W1, b1):
    raise NotImplementedError("write your pallas kernel here")


# speedup vs baseline: 28.9602x; 28.9602x over previous
import jax
import jax.numpy as jnp
from jax import lax
from jax.experimental import pallas as pl
from jax.experimental.pallas import tpu as pltpu

_N = 4096
_D = 32
_KP1 = 51
_BLK = 128
_GRID = _N // _BLK


def _body(f_ref, w0_ref, b0_ref, w1_ref, b1_ref, o_ref, emb_ref, s_ref, k_ref):
    i = pl.program_id(0)

    @pl.when(i == 0)
    def _():
        f = f_ref[...]
        h = lax.dot_general(f, w0_ref[...], (((1,), (1,)), ((), ())),
                            preferred_element_type=jnp.float32) + b0_ref[...]
        h = jnp.maximum(h, 0.0)
        h = lax.dot_general(h, w1_ref[...], (((1,), (1,)), ((), ())),
                            preferred_element_type=jnp.float32) + b1_ref[...]
        nrm = jnp.sqrt(jnp.sum(h * h, axis=1, keepdims=True))
        emb_ref[...] = h / jnp.maximum(nrm, 1e-12)

    rows = emb_ref[pl.ds(i * _BLK, _BLK), :]
    s = lax.dot_general(rows, emb_ref[...], (((1,), (1,)), ((), ())),
                        preferred_element_type=jnp.float32)
    s_ref[...] = s

    b = lax.bitcast_convert_type(s, jnp.int32)
    k_ref[...] = jnp.where(b >= 0, (b >> 1) + jnp.int32(0x40000000), (~b) >> 1)

    def step(t, prefix):
        cand = prefix | lax.shift_left(jnp.int32(1), jnp.int32(30) - t)
        cnt = jnp.sum((k_ref[...] >= cand).astype(jnp.int32), axis=1,
                      keepdims=True)
        return jnp.where(cnt >= _KP1, cand, prefix)

    prefix = lax.fori_loop(0, 31, step, jnp.zeros((_BLK, 1), jnp.int32))
    kept = k_ref[...] >= prefix
    o_ref[...] = jnp.where(kept, jnp.maximum(s_ref[...], 0.0), 0.0)


def kernel(features, W0, b0, W1, b1):
    b0r = b0.reshape(1, _D)
    b1r = b1.reshape(1, _D)
    return pl.pallas_call(
        _body,
        grid=(_GRID,),
        in_specs=[
            pl.BlockSpec((_N, _D), lambda i: (0, 0)),
            pl.BlockSpec((_D, _D), lambda i: (0, 0)),
            pl.BlockSpec((1, _D), lambda i: (0, 0)),
            pl.BlockSpec((_D, _D), lambda i: (0, 0)),
            pl.BlockSpec((1, _D), lambda i: (0, 0)),
        ],
        out_specs=pl.BlockSpec((_BLK, _N), lambda i: (i, 0)),
        out_shape=jax.ShapeDtypeStruct((_N, _N), jnp.float32),
        scratch_shapes=[
            pltpu.VMEM((_N, _D), jnp.float32),
            pltpu.VMEM((_BLK, _N), jnp.float32),
            pltpu.VMEM((_BLK, _N), jnp.int32),
        ],
    )(features, W0, b0r, W1, b1r)
